# baseline (speedup 1.0000x reference)
import functools

import jax
import jax.numpy as jnp
from jax import lax
from jax.experimental import pallas as pl
from jax.experimental.pallas import tpu as pltpu
from jax.experimental.pallas import tpu_sc as plsc

_B = 8
_N = 50000
_R = 400000
_S0 = _N // 2
_S1 = (3 * _N) // 4
_L = 16
_W = 256
_RSH = 8
_RMASK = _W - 1
_ROWS = 224
_NPAD = _ROWS * _W
_CH = 2000
_QUARTER = _R // 4
_NCHUNK = _QUARTER // _CH
_NTILES = 32

_mesh = plsc.VectorSubcoreMesh(core_axis_name="c", subcore_axis_name="s")


@functools.partial(
    pl.kernel,
    mesh=_mesh,
    out_type=(
        jax.ShapeDtypeStruct((_NTILES, _NPAD), jnp.float32),
        jax.ShapeDtypeStruct((_NTILES, 4, _L), jnp.float32),
    ),
    compiler_params=pltpu.CompilerParams(needs_layout_passes=False),
    scratch_types=[
        pltpu.VMEM((_N,), jnp.float32),
        pltpu.VMEM((_NPAD,), jnp.float32),
        pltpu.VMEM((_CH,), jnp.int32),
        pltpu.VMEM((_CH,), jnp.int32),
        pltpu.VMEM((_CH,), jnp.int32),
        pltpu.VMEM((_CH,), jnp.float32),
        pltpu.VMEM((_CH,), jnp.int32),
        pltpu.VMEM((_CH,), jnp.int32),
        pltpu.VMEM((_CH,), jnp.int32),
        pltpu.VMEM((_CH,), jnp.float32),
        pltpu.VMEM((4, _L), jnp.float32),
        pltpu.SemaphoreType.DMA,
        pltpu.SemaphoreType.DMA,
    ],
)
def _sc_three_phase(y_hbm, reac1_hbm, prod1_hbm, k1_hbm,
                    r2a_hbm, r2b_hbm, p2_hbm, k2_hbm,
                    pout_hbm, sout_hbm,
                    y_buf, p_buf,
                    i0a, i1a, i2a, f0a,
                    i0b, i1b, i2b, f0b,
                    out_v, sema, semb):
    c = lax.axis_index("c")
    s = lax.axis_index("s")
    b = c * 4 + (s // 4)
    q = s % 4
    slot = c * 16 + s
    zero16 = jnp.zeros((_L,), jnp.float32)
    iota16 = lax.iota(jnp.int32, _L)

    @plsc.parallel_loop(0, _NPAD // _L, unroll=8)
    def _zero(t):
        p_buf[pl.ds(t * _L, _L)] = zero16

    pltpu.sync_copy(y_hbm.at[b], y_buf)

    base = q * _QUARTER
    set1a = (i0a, i1a, f0a)
    set1b = (i0b, i1b, f0b)
    set2a = (i0a, i1a, i2a, f0a)
    set2b = (i0b, i1b, i2b, f0b)

    r1_src = (reac1_hbm, prod1_hbm, k1_hbm)
    r2_src = (r2a_hbm, r2b_hbm, p2_hbm, k2_hbm)

    kbase = b * _R

    def _fire(srcs, bufs, sem, off):
        for src_hbm, buf in zip(srcs, bufs):
            if src_hbm is k1_hbm or src_hbm is k2_hbm:
                pltpu.async_copy(src_hbm.at[pl.ds(kbase + off, _CH)], buf, sem)
            else:
                pltpu.async_copy(src_hbm.at[pl.ds(off, _CH)], buf, sem)

    def _drain(srcs, bufs, sem):
        for src_hbm, buf in zip(srcs, bufs):
            pltpu.make_async_copy(src_hbm.at[pl.ds(0, _CH)], buf, sem).wait()

    def _proc1(bufs, carry):
        def _inner(j, carry2):
            sg, sl = carry2
            o = j * _L
            i1 = bufs[0][pl.ds(o, _L)]
            p1 = bufs[1][pl.ds(o, _L)]
            kk = bufs[2][pl.ds(o, _L)]
            r = kk * plsc.load_gather(y_buf, [i1])
            span = jnp.uint32(_S1 - _S0)
            mp = plsc.bitcast(p1 - _S0, jnp.uint32) < span
            mi = plsc.bitcast(i1 - _S0, jnp.uint32) < span
            sg = sg + jnp.where(mp, r, 0.0)
            sl = sl + jnp.where(mi, r, 0.0)
            plsc.addupdate_scatter(p_buf, [p1], r)
            plsc.addupdate_scatter(p_buf, [i1], -r)
            return sg, sl
        return plsc.parallel_loop(0, _CH // _L, unroll=8, carry=carry)(_inner)

    def _proc2(bufs, carry):
        def _inner(j, carry2):
            sg, sl = carry2
            o = j * _L
            ia = bufs[0][pl.ds(o, _L)]
            ib = bufs[1][pl.ds(o, _L)]
            p2 = bufs[2][pl.ds(o, _L)]
            kk = bufs[3][pl.ds(o, _L)]
            r = (kk * plsc.load_gather(y_buf, [ia])
                 * plsc.load_gather(y_buf, [ib]))
            span = jnp.uint32(_S1 - _S0)
            mp = plsc.bitcast(p2 - _S0, jnp.uint32) < span
            ma = plsc.bitcast(ia - _S0, jnp.uint32) < span
            mb = plsc.bitcast(ib - _S0, jnp.uint32) < span
            sg = sg + jnp.where(mp, r, 0.0)
            sl = sl + (jnp.where(ma, r, 0.0) + jnp.where(mb, r, 0.0))
            plsc.addupdate_scatter(p_buf, [p2], r)
            plsc.addupdate_scatter(p_buf, [ia], -r)
            plsc.addupdate_scatter(p_buf, [ib], -r)
            return sg, sl
        return plsc.parallel_loop(0, _CH // _L, unroll=8, carry=carry)(_inner)

    def _phase(srcs, bufs_a, bufs_b, proc, carry):
        _fire(srcs, bufs_a, sema, base)

        def _pair(k, carry2):
            off1 = base + (2 * k + 1) * _CH
            off2 = base + jnp.minimum(2 * k + 2, _NCHUNK - 1) * _CH
            _fire(srcs, bufs_b, semb, off1)
            _drain(srcs, bufs_a, sema)
            carry2 = proc(bufs_a, carry2)
            _fire(srcs, bufs_a, sema, off2)
            _drain(srcs, bufs_b, semb)
            return proc(bufs_b, carry2)

        carry = lax.fori_loop(0, _NCHUNK // 2, _pair, carry)
        _drain(srcs, bufs_a, sema)
        return carry

    sg, sl = _phase(r1_src, set1a, set1b, _proc1, (zero16, zero16))
    sg, sl = _phase(r2_src, set2a, set2b, _proc2, (sg, sl))

    def _ysum(i, carry):
        ys, ym = carry
        v = y_buf[pl.ds(i * _L, _L)]
        ii = iota16 + i * _L
        ys = ys + jnp.where((ii >= _S0) & (ii < _S1), v, 0.0)
        ym = ym + jnp.where(ii >= _S1, v, 0.0)
        return ys, ym
    ys, ym = lax.fori_loop(_S0 // _L, _N // _L, _ysum, (zero16, zero16), unroll=8)

    out_v[0, :] = sg
    out_v[1, :] = sl
    out_v[2, :] = ys
    out_v[3, :] = ym
    pltpu.sync_copy(out_v, sout_hbm.at[slot])
    pltpu.sync_copy(p_buf, pout_hbm.at[slot])


_KR = 3125


def _tc_coeff_body(med_ref, a1_ref, g1_ref, a2_ref, g2_ref, k1_ref, k2_ref):
    bi = pl.program_id(0)
    ninvT = med_ref[bi, 0]
    den = med_ref[bi, 1]
    k1_ref[0] = a1_ref[...] * jnp.exp(g1_ref[...] * ninvT)
    k2_ref[0] = a2_ref[...] * jnp.exp(g2_ref[...] * ninvT) * den


_tc_coeff = pl.pallas_call(
    _tc_coeff_body,
    grid=(_B,),
    in_specs=[
        pl.BlockSpec((_B, 2), lambda i: (0, 0)),
        pl.BlockSpec((_KR, 128), lambda i: (0, 0)),
        pl.BlockSpec((_KR, 128), lambda i: (0, 0)),
        pl.BlockSpec((_KR, 128), lambda i: (0, 0)),
        pl.BlockSpec((_KR, 128), lambda i: (0, 0)),
    ],
    out_specs=[
        pl.BlockSpec((1, _KR, 128), lambda i: (i, 0, 0)),
        pl.BlockSpec((1, _KR, 128), lambda i: (i, 0, 0)),
    ],
    out_shape=[
        jax.ShapeDtypeStruct((_B, _KR, 128), jnp.float32),
        jax.ShapeDtypeStruct((_B, _KR, 128), jnp.float32),
    ],
)


def _tc_combine_body(s_ref, p_ref, o_ref):
    sums = s_ref[0]
    sg = jnp.sum(sums[:, 0, :])
    sl = jnp.sum(sums[:, 1, :])
    ys = jnp.sum(sums[0, 2, :])
    ym = jnp.sum(sums[0, 3, :])
    decay = jnp.minimum(2.0 / (1.0e6 * (ys + ym) + 1e-30), 1.0)
    x = (sg - sl) * 1.0e6
    smt = decay / (1.0 + jnp.exp(-x))
    o_ref[0] = smt * jnp.sum(p_ref[0], axis=0)


_tc_combine = pl.pallas_call(
    _tc_combine_body,
    grid=(_B,),
    in_specs=[
        pl.BlockSpec((1, 4, 4, _L), lambda i: (i, 0, 0, 0)),
        pl.BlockSpec((1, 4, _ROWS, _W), lambda i: (i, 0, 0, 0)),
    ],
    out_specs=pl.BlockSpec((1, _ROWS, _W), lambda i: (i, 0, 0)),
    out_shape=jax.ShapeDtypeStruct((_B, _ROWS, _W), jnp.float32),
)


def kernel(t_in, y_in, reac1, prod1, reac2a, reac2b, prod2,
           alpha1, gamma1, alpha2, gamma2):
    t32 = t_in.astype(jnp.float32)
    T = 10.0 + 5.0 * jnp.abs(jnp.sin(t32 * 1e-5))
    den = 1.0e4 * (1.0 + 0.1 * jnp.cos(t32 * 1e-5))
    med = jnp.stack([-1.0 / T, den], axis=1).astype(jnp.float32)
    i32 = lambda a: a.astype(jnp.int32)
    f32 = lambda a: a.astype(jnp.float32)
    k1, k2 = _tc_coeff(med,
                       f32(alpha1).reshape(_KR, 128),
                       f32(gamma1).reshape(_KR, 128),
                       f32(alpha2).reshape(_KR, 128),
                       f32(gamma2).reshape(_KR, 128))
    pout, sout = _sc_three_phase(
        y_in.astype(jnp.float32),
        i32(reac1), i32(prod1), k1.reshape(_B * _R),
        i32(reac2a), i32(reac2b), i32(prod2), k2.reshape(_B * _R),
    )
    dy = _tc_combine(sout.reshape(_B, 4, 4, _L),
                     pout.reshape(_B, 4, _ROWS, _W))
    return dy.reshape(_B, _NPAD)[:, :_N]


# speedup vs baseline: 1.0372x; 1.0372x over previous
import functools

import jax
import jax.numpy as jnp
from jax import lax
from jax.experimental import pallas as pl
from jax.experimental.pallas import tpu as pltpu
from jax.experimental.pallas import tpu_sc as plsc

_B = 8
_N = 50000
_R = 400000
_S0 = _N // 2
_S1 = (3 * _N) // 4
_L = 16
_W = 256
_RSH = 8
_RMASK = _W - 1
_ROWS = 224
_NPAD = _ROWS * _W
_CH = 2000
_QUARTER = _R // 4
_NCHUNK = _QUARTER // _CH
_NTILES = 32

_mesh = plsc.VectorSubcoreMesh(core_axis_name="c", subcore_axis_name="s")


@functools.partial(
    pl.kernel,
    mesh=_mesh,
    out_type=(
        jax.ShapeDtypeStruct((_NTILES, _ROWS, _W), jnp.float32),
        jax.ShapeDtypeStruct((_NTILES, 4, _L), jnp.float32),
    ),
    compiler_params=pltpu.CompilerParams(needs_layout_passes=False),
    scratch_types=[
        pltpu.VMEM((_N,), jnp.float32),
        pltpu.VMEM((_ROWS, _W), jnp.float32),
        pltpu.VMEM((_CH,), jnp.int32),
        pltpu.VMEM((_CH,), jnp.int32),
        pltpu.VMEM((_CH,), jnp.int32),
        pltpu.VMEM((_CH,), jnp.float32),
        pltpu.VMEM((_CH,), jnp.int32),
        pltpu.VMEM((_CH,), jnp.int32),
        pltpu.VMEM((_CH,), jnp.int32),
        pltpu.VMEM((_CH,), jnp.float32),
        pltpu.VMEM((4, _L), jnp.float32),
        pltpu.SemaphoreType.DMA,
        pltpu.SemaphoreType.DMA,
    ],
)
def _sc_three_phase(y_hbm, reac1_hbm, prod1_hbm, k1_hbm,
                    r2a_hbm, r2b_hbm, p2_hbm, k2_hbm,
                    pout_hbm, sout_hbm,
                    y_buf, p_buf,
                    i0a, i1a, i2a, f0a,
                    i0b, i1b, i2b, f0b,
                    out_v, sema, semb):
    c = lax.axis_index("c")
    s = lax.axis_index("s")
    b = c * 4 + (s // 4)
    q = s % 4
    slot = c * 16 + s
    zero16 = jnp.zeros((_L,), jnp.float32)
    iota16 = lax.iota(jnp.int32, _L)

    def _zero(t, carry):
        p_buf[t // (_W // _L), pl.ds((t % (_W // _L)) * _L, _L)] = zero16
        return carry
    lax.fori_loop(0, _ROWS * (_W // _L), _zero, 0, unroll=8)

    pltpu.sync_copy(y_hbm.at[b], y_buf)

    base = q * _QUARTER
    set1a = (i0a, i1a, f0a)
    set1b = (i0b, i1b, f0b)
    set2a = (i0a, i1a, i2a, f0a)
    set2b = (i0b, i1b, i2b, f0b)

    r1_src = (reac1_hbm, prod1_hbm, k1_hbm)
    r2_src = (r2a_hbm, r2b_hbm, p2_hbm, k2_hbm)

    kbase = b * _R

    def _fire(srcs, bufs, sem, off):
        for src_hbm, buf in zip(srcs, bufs):
            if src_hbm is k1_hbm or src_hbm is k2_hbm:
                pltpu.async_copy(src_hbm.at[pl.ds(kbase + off, _CH)], buf, sem)
            else:
                pltpu.async_copy(src_hbm.at[pl.ds(off, _CH)], buf, sem)

    def _drain(srcs, bufs, sem):
        for src_hbm, buf in zip(srcs, bufs):
            pltpu.make_async_copy(src_hbm.at[pl.ds(0, _CH)], buf, sem).wait()

    def _proc1(bufs, carry):
        def _inner(j, carry2):
            sg, sl = carry2
            o = j * _L
            i1 = bufs[0][pl.ds(o, _L)]
            p1 = bufs[1][pl.ds(o, _L)]
            kk = bufs[2][pl.ds(o, _L)]
            r = kk * plsc.load_gather(y_buf, [i1])
            span = jnp.uint32(_S1 - _S0)
            mp = plsc.bitcast(p1 - _S0, jnp.uint32) < span
            mi = plsc.bitcast(i1 - _S0, jnp.uint32) < span
            sg = sg + jnp.where(mp, r, 0.0)
            sl = sl + jnp.where(mi, r, 0.0)
            plsc.addupdate_scatter(p_buf, [p1 >> _RSH, p1 & _RMASK], r)
            plsc.addupdate_scatter(p_buf, [i1 >> _RSH, i1 & _RMASK], -r)
            return sg, sl
        return plsc.parallel_loop(0, _CH // _L, unroll=8, carry=carry)(_inner)

    def _proc2(bufs, carry):
        def _inner(j, carry2):
            sg, sl = carry2
            o = j * _L
            ia = bufs[0][pl.ds(o, _L)]
            ib = bufs[1][pl.ds(o, _L)]
            p2 = bufs[2][pl.ds(o, _L)]
            kk = bufs[3][pl.ds(o, _L)]
            r = (kk * plsc.load_gather(y_buf, [ia])
                 * plsc.load_gather(y_buf, [ib]))
            span = jnp.uint32(_S1 - _S0)
            mp = plsc.bitcast(p2 - _S0, jnp.uint32) < span
            ma = plsc.bitcast(ia - _S0, jnp.uint32) < span
            mb = plsc.bitcast(ib - _S0, jnp.uint32) < span
            sg = sg + jnp.where(mp, r, 0.0)
            sl = sl + (jnp.where(ma, r, 0.0) + jnp.where(mb, r, 0.0))
            plsc.addupdate_scatter(p_buf, [p2 >> _RSH, p2 & _RMASK], r)
            plsc.addupdate_scatter(p_buf, [ia >> _RSH, ia & _RMASK], -r)
            plsc.addupdate_scatter(p_buf, [ib >> _RSH, ib & _RMASK], -r)
            return sg, sl
        return plsc.parallel_loop(0, _CH // _L, unroll=8, carry=carry)(_inner)

    def _phase(srcs, bufs_a, bufs_b, proc, carry):
        _fire(srcs, bufs_a, sema, base)

        def _pair(k, carry2):
            off1 = base + (2 * k + 1) * _CH
            off2 = base + jnp.minimum(2 * k + 2, _NCHUNK - 1) * _CH
            _fire(srcs, bufs_b, semb, off1)
            _drain(srcs, bufs_a, sema)
            carry2 = proc(bufs_a, carry2)
            _fire(srcs, bufs_a, sema, off2)
            _drain(srcs, bufs_b, semb)
            return proc(bufs_b, carry2)

        carry = lax.fori_loop(0, _NCHUNK // 2, _pair, carry)
        _drain(srcs, bufs_a, sema)
        return carry

    sg, sl = _phase(r1_src, set1a, set1b, _proc1, (zero16, zero16))
    sg, sl = _phase(r2_src, set2a, set2b, _proc2, (sg, sl))

    def _ysum(i, carry):
        ys, ym = carry
        v = y_buf[pl.ds(i * _L, _L)]
        ii = iota16 + i * _L
        ys = ys + jnp.where((ii >= _S0) & (ii < _S1), v, 0.0)
        ym = ym + jnp.where(ii >= _S1, v, 0.0)
        return ys, ym
    ys, ym = lax.fori_loop(_S0 // _L, _N // _L, _ysum, (zero16, zero16), unroll=8)

    out_v[0, :] = sg
    out_v[1, :] = sl
    out_v[2, :] = ys
    out_v[3, :] = ym
    pltpu.sync_copy(out_v, sout_hbm.at[slot])
    pltpu.sync_copy(p_buf, pout_hbm.at[slot])


_KR = 3125


def _tc_coeff_body(med_ref, a1_ref, g1_ref, a2_ref, g2_ref, k1_ref, k2_ref):
    bi = pl.program_id(0)
    ninvT = med_ref[bi, 0]
    den = med_ref[bi, 1]
    k1_ref[0] = a1_ref[...] * jnp.exp(g1_ref[...] * ninvT)
    k2_ref[0] = a2_ref[...] * jnp.exp(g2_ref[...] * ninvT) * den


_tc_coeff = pl.pallas_call(
    _tc_coeff_body,
    grid=(_B,),
    in_specs=[
        pl.BlockSpec((_B, 2), lambda i: (0, 0)),
        pl.BlockSpec((_KR, 128), lambda i: (0, 0)),
        pl.BlockSpec((_KR, 128), lambda i: (0, 0)),
        pl.BlockSpec((_KR, 128), lambda i: (0, 0)),
        pl.BlockSpec((_KR, 128), lambda i: (0, 0)),
    ],
    out_specs=[
        pl.BlockSpec((1, _KR, 128), lambda i: (i, 0, 0)),
        pl.BlockSpec((1, _KR, 128), lambda i: (i, 0, 0)),
    ],
    out_shape=[
        jax.ShapeDtypeStruct((_B, _KR, 128), jnp.float32),
        jax.ShapeDtypeStruct((_B, _KR, 128), jnp.float32),
    ],
)


def _tc_combine_body(s_ref, p_ref, o_ref):
    sums = s_ref[0]
    sg = jnp.sum(sums[:, 0, :])
    sl = jnp.sum(sums[:, 1, :])
    ys = jnp.sum(sums[0, 2, :])
    ym = jnp.sum(sums[0, 3, :])
    decay = jnp.minimum(2.0 / (1.0e6 * (ys + ym) + 1e-30), 1.0)
    x = (sg - sl) * 1.0e6
    smt = decay / (1.0 + jnp.exp(-x))
    o_ref[0] = smt * jnp.sum(p_ref[0], axis=0)


_tc_combine = pl.pallas_call(
    _tc_combine_body,
    grid=(_B,),
    in_specs=[
        pl.BlockSpec((1, 4, 4, _L), lambda i: (i, 0, 0, 0)),
        pl.BlockSpec((1, 4, _ROWS, _W), lambda i: (i, 0, 0, 0)),
    ],
    out_specs=pl.BlockSpec((1, _ROWS, _W), lambda i: (i, 0, 0)),
    out_shape=jax.ShapeDtypeStruct((_B, _ROWS, _W), jnp.float32),
)


def kernel(t_in, y_in, reac1, prod1, reac2a, reac2b, prod2,
           alpha1, gamma1, alpha2, gamma2):
    t32 = t_in.astype(jnp.float32)
    T = 10.0 + 5.0 * jnp.abs(jnp.sin(t32 * 1e-5))
    den = 1.0e4 * (1.0 + 0.1 * jnp.cos(t32 * 1e-5))
    med = jnp.stack([-1.0 / T, den], axis=1).astype(jnp.float32)
    i32 = lambda a: a.astype(jnp.int32)
    f32 = lambda a: a.astype(jnp.float32)
    k1, k2 = _tc_coeff(med,
                       f32(alpha1).reshape(_KR, 128),
                       f32(gamma1).reshape(_KR, 128),
                       f32(alpha2).reshape(_KR, 128),
                       f32(gamma2).reshape(_KR, 128))
    pout, sout = _sc_three_phase(
        y_in.astype(jnp.float32),
        i32(reac1), i32(prod1), k1.reshape(_B * _R),
        i32(reac2a), i32(reac2b), i32(prod2), k2.reshape(_B * _R),
    )
    dy = _tc_combine(sout.reshape(_B, 4, 4, _L),
                     pout.reshape(_B, 4, _ROWS, _W))
    return dy.reshape(_B, _NPAD)[:, :_N]


# speedup vs baseline: 1.1323x; 1.0917x over previous
import functools

import jax
import jax.numpy as jnp
from jax import lax
from jax.experimental import pallas as pl
from jax.experimental.pallas import tpu as pltpu
from jax.experimental.pallas import tpu_sc as plsc

_B = 8
_N = 50000
_R = 400000
_S0 = _N // 2
_S1 = (3 * _N) // 4
_L = 16
_W = 256
_RSH = 8
_RMASK = _W - 1
_ROWS = 224
_NPAD = _ROWS * _W
_CH = 2000
_QUARTER = _R // 4
_NCHUNK = _QUARTER // _CH
_NTILES = 32

_mesh = plsc.VectorSubcoreMesh(core_axis_name="c", subcore_axis_name="s")


@functools.partial(
    pl.kernel,
    mesh=_mesh,
    out_type=(
        jax.ShapeDtypeStruct((_NTILES, _ROWS, _W), jnp.float32),
        jax.ShapeDtypeStruct((_NTILES, 2, _L), jnp.float32),
    ),
    compiler_params=pltpu.CompilerParams(needs_layout_passes=False),
    scratch_types=[
        pltpu.VMEM((_N,), jnp.float32),
        pltpu.VMEM((_ROWS, _W), jnp.float32),
        pltpu.VMEM((_CH,), jnp.int32),
        pltpu.VMEM((_CH,), jnp.int32),
        pltpu.VMEM((_CH,), jnp.int32),
        pltpu.VMEM((_CH,), jnp.float32),
        pltpu.VMEM((_CH,), jnp.int32),
        pltpu.VMEM((_CH,), jnp.int32),
        pltpu.VMEM((_CH,), jnp.int32),
        pltpu.VMEM((_CH,), jnp.float32),
        pltpu.VMEM((2, _L), jnp.float32),
        pltpu.SemaphoreType.DMA,
        pltpu.SemaphoreType.DMA,
    ],
)
def _sc_three_phase(y_hbm, reac1_hbm, prod1_hbm, k1_hbm,
                    r2a_hbm, r2b_hbm, p2_hbm, k2_hbm,
                    pout_hbm, sout_hbm,
                    y_buf, p_buf,
                    i0a, i1a, i2a, f0a,
                    i0b, i1b, i2b, f0b,
                    out_v, sema, semb):
    c = lax.axis_index("c")
    s = lax.axis_index("s")
    b = c * 4 + (s // 4)
    q = s % 4
    slot = c * 16 + s
    zero16 = jnp.zeros((_L,), jnp.float32)
    iota16 = lax.iota(jnp.int32, _L)

    def _zero(t, carry):
        p_buf[t // (_W // _L), pl.ds((t % (_W // _L)) * _L, _L)] = zero16
        return carry
    lax.fori_loop(0, _ROWS * (_W // _L), _zero, 0, unroll=8)

    pltpu.sync_copy(y_hbm.at[b], y_buf)

    base = q * _QUARTER
    set1a = (i0a, i1a, f0a)
    set1b = (i0b, i1b, f0b)
    set2a = (i0a, i1a, i2a, f0a)
    set2b = (i0b, i1b, i2b, f0b)

    r1_src = (reac1_hbm, prod1_hbm, k1_hbm)
    r2_src = (r2a_hbm, r2b_hbm, p2_hbm, k2_hbm)

    kbase = b * _R

    def _fire(srcs, bufs, sem, off):
        for src_hbm, buf in zip(srcs, bufs):
            if src_hbm is k1_hbm or src_hbm is k2_hbm:
                pltpu.async_copy(src_hbm.at[pl.ds(kbase + off, _CH)], buf, sem)
            else:
                pltpu.async_copy(src_hbm.at[pl.ds(off, _CH)], buf, sem)

    def _drain(srcs, bufs, sem):
        for src_hbm, buf in zip(srcs, bufs):
            pltpu.make_async_copy(src_hbm.at[pl.ds(0, _CH)], buf, sem).wait()

    def _proc1(bufs):
        @plsc.parallel_loop(0, _CH // _L, unroll=8)
        def _inner(j):
            o = j * _L
            i1 = bufs[0][pl.ds(o, _L)]
            p1 = bufs[1][pl.ds(o, _L)]
            kk = bufs[2][pl.ds(o, _L)]
            r = kk * plsc.load_gather(y_buf, [i1])
            plsc.addupdate_scatter(p_buf, [p1 >> _RSH, p1 & _RMASK], r)
            plsc.addupdate_scatter(p_buf, [i1 >> _RSH, i1 & _RMASK], -r)

    def _proc2(bufs):
        @plsc.parallel_loop(0, _CH // _L, unroll=8)
        def _inner(j):
            o = j * _L
            ia = bufs[0][pl.ds(o, _L)]
            ib = bufs[1][pl.ds(o, _L)]
            p2 = bufs[2][pl.ds(o, _L)]
            kk = bufs[3][pl.ds(o, _L)]
            r = (kk * plsc.load_gather(y_buf, [ia])
                 * plsc.load_gather(y_buf, [ib]))
            plsc.addupdate_scatter(p_buf, [p2 >> _RSH, p2 & _RMASK], r)
            plsc.addupdate_scatter(p_buf, [ia >> _RSH, ia & _RMASK], -r)
            plsc.addupdate_scatter(p_buf, [ib >> _RSH, ib & _RMASK], -r)

    def _phase(srcs, bufs_a, bufs_b, proc):
        _fire(srcs, bufs_a, sema, base)

        def _pair(k, carry2):
            off1 = base + (2 * k + 1) * _CH
            off2 = base + jnp.minimum(2 * k + 2, _NCHUNK - 1) * _CH
            _fire(srcs, bufs_b, semb, off1)
            _drain(srcs, bufs_a, sema)
            proc(bufs_a)
            _fire(srcs, bufs_a, sema, off2)
            _drain(srcs, bufs_b, semb)
            proc(bufs_b)
            return carry2

        lax.fori_loop(0, _NCHUNK // 2, _pair, 0)
        _drain(srcs, bufs_a, sema)

    _phase(r1_src, set1a, set1b, _proc1)
    _phase(r2_src, set2a, set2b, _proc2)

    def _ysum(i, carry):
        ys, ym = carry
        v = y_buf[pl.ds(i * _L, _L)]
        ii = iota16 + i * _L
        ys = ys + jnp.where((ii >= _S0) & (ii < _S1), v, 0.0)
        ym = ym + jnp.where(ii >= _S1, v, 0.0)
        return ys, ym
    ys, ym = lax.fori_loop(_S0 // _L, _N // _L, _ysum, (zero16, zero16), unroll=8)

    out_v[0, :] = ys
    out_v[1, :] = ym
    pltpu.sync_copy(out_v, sout_hbm.at[slot])
    pltpu.sync_copy(p_buf, pout_hbm.at[slot])


_KR = 3125


def _tc_coeff_body(med_ref, a1_ref, g1_ref, a2_ref, g2_ref, k1_ref, k2_ref):
    bi = pl.program_id(0)
    ninvT = med_ref[bi, 0]
    den = med_ref[bi, 1]
    k1_ref[0] = a1_ref[...] * jnp.exp(g1_ref[...] * ninvT)
    k2_ref[0] = a2_ref[...] * jnp.exp(g2_ref[...] * ninvT) * den


_tc_coeff = pl.pallas_call(
    _tc_coeff_body,
    grid=(_B,),
    in_specs=[
        pl.BlockSpec((_B, 2), lambda i: (0, 0)),
        pl.BlockSpec((_KR, 128), lambda i: (0, 0)),
        pl.BlockSpec((_KR, 128), lambda i: (0, 0)),
        pl.BlockSpec((_KR, 128), lambda i: (0, 0)),
        pl.BlockSpec((_KR, 128), lambda i: (0, 0)),
    ],
    out_specs=[
        pl.BlockSpec((1, _KR, 128), lambda i: (i, 0, 0)),
        pl.BlockSpec((1, _KR, 128), lambda i: (i, 0, 0)),
    ],
    out_shape=[
        jax.ShapeDtypeStruct((_B, _KR, 128), jnp.float32),
        jax.ShapeDtypeStruct((_B, _KR, 128), jnp.float32),
    ],
)


def _tc_combine_body(s_ref, p_ref, o_ref):
    sums = s_ref[0]
    ys = jnp.sum(sums[0, 0, :])
    ym = jnp.sum(sums[0, 1, :])
    psum = jnp.sum(p_ref[0], axis=0)
    n = (lax.broadcasted_iota(jnp.int32, (_ROWS, _W), 0) * _W
         + lax.broadcasted_iota(jnp.int32, (_ROWS, _W), 1))
    m = (n >= _S0) & (n < _S1)
    x = jnp.sum(jnp.where(m, psum, 0.0)) * 1.0e6
    decay = jnp.minimum(2.0 / (1.0e6 * (ys + ym) + 1e-30), 1.0)
    smt = decay / (1.0 + jnp.exp(-x))
    o_ref[0] = smt * psum


_tc_combine = pl.pallas_call(
    _tc_combine_body,
    grid=(_B,),
    in_specs=[
        pl.BlockSpec((1, 4, 2, _L), lambda i: (i, 0, 0, 0)),
        pl.BlockSpec((1, 4, _ROWS, _W), lambda i: (i, 0, 0, 0)),
    ],
    out_specs=pl.BlockSpec((1, _ROWS, _W), lambda i: (i, 0, 0)),
    out_shape=jax.ShapeDtypeStruct((_B, _ROWS, _W), jnp.float32),
)


def kernel(t_in, y_in, reac1, prod1, reac2a, reac2b, prod2,
           alpha1, gamma1, alpha2, gamma2):
    t32 = t_in.astype(jnp.float32)
    T = 10.0 + 5.0 * jnp.abs(jnp.sin(t32 * 1e-5))
    den = 1.0e4 * (1.0 + 0.1 * jnp.cos(t32 * 1e-5))
    med = jnp.stack([-1.0 / T, den], axis=1).astype(jnp.float32)
    i32 = lambda a: a.astype(jnp.int32)
    f32 = lambda a: a.astype(jnp.float32)
    k1, k2 = _tc_coeff(med,
                       f32(alpha1).reshape(_KR, 128),
                       f32(gamma1).reshape(_KR, 128),
                       f32(alpha2).reshape(_KR, 128),
                       f32(gamma2).reshape(_KR, 128))
    pout, sout = _sc_three_phase(
        y_in.astype(jnp.float32),
        i32(reac1), i32(prod1), k1.reshape(_B * _R),
        i32(reac2a), i32(reac2b), i32(prod2), k2.reshape(_B * _R),
    )
    dy = _tc_combine(sout.reshape(_B, 4, 2, _L),
                     pout.reshape(_B, 4, _ROWS, _W))
    return dy.reshape(_B, _NPAD)[:, :_N]
